# Initial kernel scaffold; baseline (speedup 1.0000x reference)
import jax
import jax.numpy as jnp
from jax.experimental import pallas as pl


def kernel(x, edge_index, edge_attr, x_emb1, x_emb2, W1, b1, W2, b2, e_emb1, e_emb2, gamma, beta):
    raise NotImplementedError("write your pallas kernel here")


# speedup vs baseline: 1.1708x; 1.1708x over previous
import functools

import jax
import jax.numpy as jnp
from jax import lax
from jax.experimental import pallas as pl
from jax.experimental.pallas import tpu as pltpu
from jax.experimental.pallas import tpu_sc as plsc

N = 10000
EMB = 256
E = 160000
SE = E + N
NSUB = 16
NCORE = 2
NW = NCORE * NSUB
CH = 128
NCHUNK = 42
EPW = NCHUNK * CH
EPAD = EPW * NW
BLK = 1000
NBLK = N // BLK
MBLK = SE // BLK


def _gather_body(h_hbm, src_hbm, out_hbm, idx_v, rows_v, sem):
    c = lax.axis_index("c")
    s = lax.axis_index("s")
    w = c * NSUB + s
    pltpu.sync_copy(src_hbm.at[w], idx_v)
    base = w * EPW

    def chunk(j, _):
        pltpu.async_copy(h_hbm.at[idx_v.at[j]], rows_v, sem).wait()
        pltpu.sync_copy(rows_v, out_hbm.at[pl.ds(base + j * CH, CH)])
        return 0
    lax.fori_loop(0, NCHUNK, chunk, 0)


@functools.lru_cache(maxsize=1)
def _sc_gather():
    mesh = plsc.VectorSubcoreMesh(
        core_axis_name="c", subcore_axis_name="s", num_cores=NCORE,
        num_subcores=NSUB)
    return functools.partial(
        pl.kernel,
        out_type=jax.ShapeDtypeStruct((EPAD, EMB), jnp.float32),
        mesh=mesh,
        scratch_types=[
            pltpu.VMEM((NCHUNK, CH), jnp.int32),
            pltpu.VMEM((CH, EMB), jnp.float32),
            pltpu.SemaphoreType.DMA,
        ],
    )(_gather_body)


def _embed_body(x0_ref, x1_ref, M_ref, out_ref):
    idx = (x0_ref[0, 0, :] * 3 + x1_ref[0, 0, :])[:, None]
    oh = (idx == lax.broadcasted_iota(jnp.int32, (1, 360), 1)).astype(jnp.float32)
    out_ref[...] = jnp.dot(oh, M_ref[...], preferred_element_type=jnp.float32,
                           precision=lax.Precision.HIGHEST)


_embed = pl.pallas_call(
    _embed_body,
    grid=(NBLK,),
    in_specs=[
        pl.BlockSpec((1, 1, BLK), lambda b: (b, 0, 0)),
        pl.BlockSpec((1, 1, BLK), lambda b: (b, 0, 0)),
        pl.BlockSpec((360, EMB), lambda b: (0, 0)),
    ],
    out_specs=pl.BlockSpec((BLK, EMB), lambda b: (b, 0)),
    out_shape=jax.ShapeDtypeStruct((N, EMB), jnp.float32),
)


def _msg_body(rows_ref, c_ref, T_ref, out_ref):
    cidx = c_ref[0, 0, :][:, None]
    oh = (cidx == lax.broadcasted_iota(jnp.int32, (1, 18), 1)).astype(jnp.float32)
    emb = jnp.dot(oh, T_ref[...], preferred_element_type=jnp.float32,
                  precision=lax.Precision.HIGHEST)
    out_ref[...] = rows_ref[...] + emb


_msg = pl.pallas_call(
    _msg_body,
    grid=(MBLK,),
    in_specs=[
        pl.BlockSpec((BLK, EMB), lambda b: (b, 0)),
        pl.BlockSpec((1, 1, BLK), lambda b: (b, 0, 0)),
        pl.BlockSpec((18, EMB), lambda b: (0, 0)),
    ],
    out_specs=pl.BlockSpec((BLK, EMB), lambda b: (b, 0)),
    out_shape=jax.ShapeDtypeStruct((SE, EMB), jnp.float32),
)


def _mlp_body(agg_ref, W1, b1, W2, b2, hn_ref):
    a_bf = agg_ref[...].astype(jnp.bfloat16)
    hid = jnp.maximum(
        jnp.dot(a_bf, W1[...].astype(jnp.bfloat16),
                preferred_element_type=jnp.float32) + b1[...], 0.0)
    hn_ref[...] = jnp.dot(hid.astype(jnp.bfloat16),
                          W2[...].astype(jnp.bfloat16),
                          preferred_element_type=jnp.float32) + b2[...]


_mlp = pl.pallas_call(
    _mlp_body,
    grid=(NBLK,),
    in_specs=[
        pl.BlockSpec((BLK, EMB), lambda b: (b, 0)),
        pl.BlockSpec((EMB, 2 * EMB), lambda b: (0, 0)),
        pl.BlockSpec((1, 2 * EMB), lambda b: (0, 0)),
        pl.BlockSpec((2 * EMB, EMB), lambda b: (0, 0)),
        pl.BlockSpec((1, EMB), lambda b: (0, 0)),
    ],
    out_specs=pl.BlockSpec((BLK, EMB), lambda b: (b, 0)),
    out_shape=jax.ShapeDtypeStruct((N, EMB), jnp.float32),
)


def _tree8(a):
    t = a[0:4, :] + a[4:8, :]
    t2 = t[0:2, :] + t[2:4, :]
    return t2[0:1, :] + t2[1:2, :]


def _colsum(x_ref, sel):
    nb8 = N // 8
    half = nb8 // 2

    def mk(lo):
        def step(i, acc):
            return acc + sel(x_ref[pl.ds((lo + i) * 8, 8), :])
        return lax.fori_loop(0, half, step, jnp.zeros((8, EMB), jnp.float32))
    return _tree8(mk(0)) + _tree8(mk(half))


def _bn_body(hn_ref, g_ref, bt_ref, out_ref, relu):
    rN = jnp.float32(1.0 / N)
    mean = _colsum(hn_ref, lambda v: v) * rN
    d = hn_ref[...] - mean

    def selsq(v):
        t = v - mean
        return t * t
    var = _colsum(hn_ref, selsq) * rN
    v = d / jnp.sqrt(var + 1e-5) * g_ref[...] + bt_ref[...]
    if relu:
        v = jnp.maximum(v, 0.0)
    out_ref[...] = v


def _mk_bn(relu):
    return pl.pallas_call(
        functools.partial(_bn_body, relu=relu),
        in_specs=[
            pl.BlockSpec((N, EMB), lambda: (0, 0)),
            pl.BlockSpec((1, EMB), lambda: (0, 0)),
            pl.BlockSpec((1, EMB), lambda: (0, 0)),
        ],
        out_specs=pl.BlockSpec((N, EMB), lambda: (0, 0)),
        out_shape=jax.ShapeDtypeStruct((N, EMB), jnp.float32),
    )


_bn = _mk_bn(True)
_bnf = _mk_bn(False)


def kernel(x, edge_index, edge_attr, x_emb1, x_emb2, W1, b1, W2, b2,
           e_emb1, e_emb2, gamma, beta):
    num_layer = W1.shape[0]
    sl = jnp.arange(N, dtype=jnp.int32)
    src_full = jnp.concatenate([edge_index[0].astype(jnp.int32), sl])
    dst_full = jnp.concatenate([edge_index[1].astype(jnp.int32), sl])
    src_pad = jnp.concatenate(
        [src_full, jnp.zeros((EPAD - SE,), jnp.int32)]).reshape(NW, NCHUNK, CH)
    c_full = jnp.concatenate(
        [(edge_attr[:, 0] * 3 + edge_attr[:, 1]).astype(jnp.int32),
         jnp.full((N,), 12, jnp.int32)]).reshape(MBLK, 1, BLK)
    x0r = x[:, 0].astype(jnp.int32).reshape(NBLK, 1, BLK)
    x1r = x[:, 1].astype(jnp.int32).reshape(NBLK, 1, BLK)

    j = jnp.arange(360, dtype=jnp.int32)
    M = x_emb1[j // 3] + x_emb2[j % 3]
    cidx = jnp.arange(18, dtype=jnp.int32)
    T = e_emb1[:, cidx // 3, :] + e_emb2[:, cidx % 3, :]

    gather = _sc_gather()
    h = _embed(x0r, x1r, M)
    for l in range(num_layer):
        rows = gather(h, src_pad)
        msg = _msg(rows, c_full, T[l])
        agg = jax.ops.segment_sum(msg, dst_full, num_segments=N)
        hn = _mlp(agg, W1[l], b1[l][None, :], W2[l], b2[l][None, :])
        if l != num_layer - 1:
            h = _bn(hn, gamma[l][None, :], beta[l][None, :])
        else:
            h = _bnf(hn, gamma[l][None, :], beta[l][None, :])
    return h


# speedup vs baseline: 1.2004x; 1.0252x over previous
import functools

import jax
import jax.numpy as jnp
from jax import lax
from jax.experimental import pallas as pl
from jax.experimental.pallas import tpu as pltpu
from jax.experimental.pallas import tpu_sc as plsc

N = 10000
EMB = 256
E = 160000
SE = E + N
NSUB = 16
NCORE = 2
NW = NCORE * NSUB
CH = 128
NCHUNK = 42
EPW = NCHUNK * CH
EPAD = EPW * NW
BLK = 1000
NBLK = N // BLK
MBLK = SE // BLK


def _gather_body(h_hbm, src_hbm, out_hbm, idx_v, rows_a, rows_b, sem_a, sem_b):
    c = lax.axis_index("c")
    s = lax.axis_index("s")
    w = c * NSUB + s
    pltpu.sync_copy(src_hbm.at[w], idx_v)
    base = w * EPW

    pltpu.async_copy(h_hbm.at[idx_v.at[0]], rows_a, sem_a)

    def pair(p, _):
        j0 = 2 * p
        j1 = j0 + 1
        pltpu.async_copy(h_hbm.at[idx_v.at[j1]], rows_b, sem_b)
        pltpu.make_async_copy(h_hbm.at[idx_v.at[j0]], rows_a, sem_a).wait()
        pltpu.sync_copy(rows_a, out_hbm.at[pl.ds(base + j0 * CH, CH)])

        @pl.when(j0 + 2 < NCHUNK)
        def _():
            pltpu.async_copy(h_hbm.at[idx_v.at[j0 + 2]], rows_a, sem_a)
        pltpu.make_async_copy(h_hbm.at[idx_v.at[j1]], rows_b, sem_b).wait()
        pltpu.sync_copy(rows_b, out_hbm.at[pl.ds(base + j1 * CH, CH)])
        return 0
    lax.fori_loop(0, NCHUNK // 2, pair, 0)


@functools.lru_cache(maxsize=1)
def _sc_gather():
    mesh = plsc.VectorSubcoreMesh(
        core_axis_name="c", subcore_axis_name="s", num_cores=NCORE,
        num_subcores=NSUB)
    return functools.partial(
        pl.kernel,
        out_type=jax.ShapeDtypeStruct((EPAD, EMB), jnp.float32),
        mesh=mesh,
        scratch_types=[
            pltpu.VMEM((NCHUNK, CH), jnp.int32),
            pltpu.VMEM((CH, EMB), jnp.float32),
            pltpu.VMEM((CH, EMB), jnp.float32),
            pltpu.SemaphoreType.DMA,
            pltpu.SemaphoreType.DMA,
        ],
    )(_gather_body)


def _embed_body(x0_ref, x1_ref, M_ref, out_ref):
    idx = (x0_ref[0, 0, :] * 3 + x1_ref[0, 0, :])[:, None]
    oh = (idx == lax.broadcasted_iota(jnp.int32, (1, 360), 1)).astype(jnp.float32)
    out_ref[...] = jnp.dot(oh, M_ref[...], preferred_element_type=jnp.float32,
                           precision=lax.Precision.HIGHEST)


_embed = pl.pallas_call(
    _embed_body,
    grid=(NBLK,),
    in_specs=[
        pl.BlockSpec((1, 1, BLK), lambda b: (b, 0, 0)),
        pl.BlockSpec((1, 1, BLK), lambda b: (b, 0, 0)),
        pl.BlockSpec((360, EMB), lambda b: (0, 0)),
    ],
    out_specs=pl.BlockSpec((BLK, EMB), lambda b: (b, 0)),
    out_shape=jax.ShapeDtypeStruct((N, EMB), jnp.float32),
)


def _msg_body(rows_ref, c_ref, T_ref, out_ref):
    cidx = c_ref[0, 0, :][:, None]
    oh = (cidx == lax.broadcasted_iota(jnp.int32, (1, 18), 1)).astype(jnp.float32)
    emb = jnp.dot(oh, T_ref[...], preferred_element_type=jnp.float32,
                  precision=lax.Precision.HIGHEST)
    out_ref[...] = rows_ref[...] + emb


_msg = pl.pallas_call(
    _msg_body,
    grid=(MBLK,),
    in_specs=[
        pl.BlockSpec((BLK, EMB), lambda b: (b, 0)),
        pl.BlockSpec((1, 1, BLK), lambda b: (b, 0, 0)),
        pl.BlockSpec((18, EMB), lambda b: (0, 0)),
    ],
    out_specs=pl.BlockSpec((BLK, EMB), lambda b: (b, 0)),
    out_shape=jax.ShapeDtypeStruct((SE, EMB), jnp.float32),
)


def _mlp_body(agg_ref, W1, b1, W2, b2, hn_ref):
    a_bf = agg_ref[...].astype(jnp.bfloat16)
    hid = jnp.maximum(
        jnp.dot(a_bf, W1[...].astype(jnp.bfloat16),
                preferred_element_type=jnp.float32) + b1[...], 0.0)
    hn_ref[...] = jnp.dot(hid.astype(jnp.bfloat16),
                          W2[...].astype(jnp.bfloat16),
                          preferred_element_type=jnp.float32) + b2[...]


_mlp = pl.pallas_call(
    _mlp_body,
    grid=(NBLK,),
    in_specs=[
        pl.BlockSpec((BLK, EMB), lambda b: (b, 0)),
        pl.BlockSpec((EMB, 2 * EMB), lambda b: (0, 0)),
        pl.BlockSpec((1, 2 * EMB), lambda b: (0, 0)),
        pl.BlockSpec((2 * EMB, EMB), lambda b: (0, 0)),
        pl.BlockSpec((1, EMB), lambda b: (0, 0)),
    ],
    out_specs=pl.BlockSpec((BLK, EMB), lambda b: (b, 0)),
    out_shape=jax.ShapeDtypeStruct((N, EMB), jnp.float32),
)


def _tree8(a):
    t = a[0:4, :] + a[4:8, :]
    t2 = t[0:2, :] + t[2:4, :]
    return t2[0:1, :] + t2[1:2, :]


def _colsum(x_ref, sel):
    nb8 = N // 8
    half = nb8 // 2

    def mk(lo):
        def step(i, acc):
            return acc + sel(x_ref[pl.ds((lo + i) * 8, 8), :])
        return lax.fori_loop(0, half, step, jnp.zeros((8, EMB), jnp.float32))
    return _tree8(mk(0)) + _tree8(mk(half))


def _bn_body(hn_ref, g_ref, bt_ref, out_ref, relu):
    rN = jnp.float32(1.0 / N)
    mean = _colsum(hn_ref, lambda v: v) * rN
    d = hn_ref[...] - mean

    def selsq(v):
        t = v - mean
        return t * t
    var = _colsum(hn_ref, selsq) * rN
    v = d / jnp.sqrt(var + 1e-5) * g_ref[...] + bt_ref[...]
    if relu:
        v = jnp.maximum(v, 0.0)
    out_ref[...] = v


def _mk_bn(relu):
    return pl.pallas_call(
        functools.partial(_bn_body, relu=relu),
        in_specs=[
            pl.BlockSpec((N, EMB), lambda: (0, 0)),
            pl.BlockSpec((1, EMB), lambda: (0, 0)),
            pl.BlockSpec((1, EMB), lambda: (0, 0)),
        ],
        out_specs=pl.BlockSpec((N, EMB), lambda: (0, 0)),
        out_shape=jax.ShapeDtypeStruct((N, EMB), jnp.float32),
    )


_bn = _mk_bn(True)
_bnf = _mk_bn(False)


def kernel(x, edge_index, edge_attr, x_emb1, x_emb2, W1, b1, W2, b2,
           e_emb1, e_emb2, gamma, beta):
    num_layer = W1.shape[0]
    sl = jnp.arange(N, dtype=jnp.int32)
    src_full = jnp.concatenate([edge_index[0].astype(jnp.int32), sl])
    dst_full = jnp.concatenate([edge_index[1].astype(jnp.int32), sl])
    src_pad = jnp.concatenate(
        [src_full, jnp.zeros((EPAD - SE,), jnp.int32)]).reshape(NW, NCHUNK, CH)
    c_full = jnp.concatenate(
        [(edge_attr[:, 0] * 3 + edge_attr[:, 1]).astype(jnp.int32),
         jnp.full((N,), 12, jnp.int32)]).reshape(MBLK, 1, BLK)
    x0r = x[:, 0].astype(jnp.int32).reshape(NBLK, 1, BLK)
    x1r = x[:, 1].astype(jnp.int32).reshape(NBLK, 1, BLK)

    j = jnp.arange(360, dtype=jnp.int32)
    M = x_emb1[j // 3] + x_emb2[j % 3]
    cidx = jnp.arange(18, dtype=jnp.int32)
    T = e_emb1[:, cidx // 3, :] + e_emb2[:, cidx % 3, :]

    gather = _sc_gather()
    h = _embed(x0r, x1r, M)
    for l in range(num_layer):
        rows = gather(h, src_pad)
        msg = _msg(rows, c_full, T[l])
        agg = jax.ops.segment_sum(msg, dst_full, num_segments=N)
        hn = _mlp(agg, W1[l], b1[l][None, :], W2[l], b2[l][None, :])
        if l != num_layer - 1:
            h = _bn(hn, gamma[l][None, :], beta[l][None, :])
        else:
            h = _bnf(hn, gamma[l][None, :], beta[l][None, :])
    return h
